# baseline (speedup 1.0000x reference)
import functools

import jax
import jax.numpy as jnp
from jax import lax
from jax.experimental import pallas as pl
from jax.experimental.pallas import tpu as pltpu
from jax.experimental.pallas import tpu_sc as plsc

B = 16384
D = 16
NC = 2
NS = 16
NW = NC * NS
BPW = B // NW
CH = 128
NCH = BPW // CH
NG = BPW // 16
EPS2 = 1e-16


def _rsqrt(x):
    i = lax.bitcast_convert_type(x, jnp.int32)
    i = jnp.int32(0x5F3759DF) - lax.shift_right_logical(i, 1)
    y = lax.bitcast_convert_type(i, jnp.float32)
    for _ in range(3):
        y = y * (1.5 - 0.5 * x * y * y)
    return y


def _body(uidx_hbm, midx_hbm, utab_hbm, mtab_hbm, out_hbm,
          uidx_v, midx_v, urows_v, mrows_v, out_v, sem):
    wid = lax.axis_index("s") * NC + lax.axis_index("c")
    base = wid * BPW

    for j in range(NCH):
        pltpu.sync_copy(uidx_hbm.at[pl.ds(base + j * CH, CH)], uidx_v.at[j])
        pltpu.sync_copy(midx_hbm.at[pl.ds(base + j * CH, CH)], midx_v.at[j])

    copies = []
    for j in range(NCH):
        copies.append(pltpu.async_copy(
            utab_hbm.at[uidx_v.at[j]], urows_v.at[pl.ds(j * CH, CH)], sem))
        copies.append(pltpu.async_copy(
            mtab_hbm.at[midx_v.at[j]], mrows_v.at[pl.ds(j * CH, CH)], sem))
    for c in copies:
        c.wait()

    lanes = lax.iota(jnp.int32, 16)

    def group(g, carry):
        ridx = g * 16 + lanes
        acc_um = jnp.zeros((16,), jnp.float32)
        acc_uu = jnp.zeros((16,), jnp.float32)
        acc_mm = jnp.zeros((16,), jnp.float32)
        for d in range(D):
            didx = jnp.full((16,), d, jnp.int32)
            u = plsc.load_gather(urows_v, [ridx, didx])
            m = plsc.load_gather(mrows_v, [ridx, didx])
            acc_um = acc_um + u * m
            acc_uu = acc_uu + u * u
            acc_mm = acc_mm + m * m
        denom2 = jnp.maximum(acc_uu, EPS2) * jnp.maximum(acc_mm, EPS2)
        sim = acc_um * _rsqrt(denom2) * 2.5 + 2.75
        out_v[pl.ds(g * 16, 16)] = sim
        return carry

    lax.fori_loop(0, NG, group, jnp.int32(0))

    pltpu.sync_copy(out_v, out_hbm.at[pl.ds(base, BPW)])


_mesh = plsc.VectorSubcoreMesh(core_axis_name="c", subcore_axis_name="s")

_sc_call = functools.partial(
    pl.kernel,
    mesh=_mesh,
    compiler_params=pltpu.CompilerParams(
        needs_layout_passes=False, use_tc_tiling_on_sc=False),
    out_type=jax.ShapeDtypeStruct((B,), jnp.float32),
    scratch_types=[
        pltpu.VMEM((NCH, CH), jnp.int32),
        pltpu.VMEM((NCH, CH), jnp.int32),
        pltpu.VMEM((BPW, D), jnp.float32),
        pltpu.VMEM((BPW, D), jnp.float32),
        pltpu.VMEM((BPW,), jnp.float32),
        pltpu.SemaphoreType.DMA,
    ],
)(_body)


def kernel(user_idx, movie_idx, user_table, movie_table):
    return _sc_call(user_idx.astype(jnp.int32), movie_idx.astype(jnp.int32),
                    user_table, movie_table)
